# Initial kernel scaffold; baseline (speedup 1.0000x reference)
import jax
import jax.numpy as jnp
from jax.experimental import pallas as pl


def kernel(node_feat, edge_index, W0, b0, W1, b1, W2, b2, W3, b3, Wc1, bc1, Wc2, bc2, Wd, bd):
    raise NotImplementedError("write your pallas kernel here")


# speedup vs baseline: 6.7800x; 6.7800x over previous
import functools

import jax
import jax.numpy as jnp
from jax import lax
from jax.experimental import pallas as pl
from jax.experimental.pallas import tpu as pltpu
from jax.experimental.pallas import tpu_sc as plsc

N = 10000
E = 320000
D = 128
G = 100
NPG = 100
K = 30
TL = 97

NC, NS = 2, 16
NW = NC * NS
CHUNK = 128
NCHUNK = -(-E // (NW * CHUNK))
EPW = NCHUNK * CHUNK
E_PAD = EPW * NW
N_PAD = 10240
RPS = N_PAD // NS


def _edge_agg_body(with_deg, d, h_hbm, src_hbm, dst_hbm, zeros_hbm, zeros1_hbm,
                   *refs):
    dacc = None
    if with_deg:
        agg_out, deg_out, src_v, dst_v, rows_v, ones_v, accum, dacc, sem = refs
    else:
        agg_out, src_v, dst_v, rows_v, accum, sem = refs
    c = lax.axis_index("c")
    s = lax.axis_index("s")
    wid = s * NC + c

    if True:
        pltpu.sync_copy(zeros_hbm.at[pl.ds(s * RPS, RPS)],
                        accum.at[pl.ds(s * RPS, RPS)])
        if with_deg:
            pltpu.sync_copy(zeros1_hbm.at[pl.ds(s * RPS, RPS)],
                            dacc.at[pl.ds(s * RPS, RPS)])
            for t in range(CHUNK // 16):
                ones_v[pl.ds(t * 16, 16)] = jnp.full((16,), 1.0, jnp.float32)
        plsc.subcore_barrier()

        pltpu.sync_copy(src_hbm.at[wid], src_v)
        pltpu.sync_copy(dst_hbm.at[wid], dst_v)

        def chunk_body(j, carry):
            pltpu.async_copy(h_hbm.at[src_v.at[j]], rows_v, sem).wait()
            pltpu.sync_copy(rows_v, accum.at[dst_v.at[j]], add=True)
            if with_deg:
                pltpu.sync_copy(ones_v, dacc.at[dst_v.at[j]], add=True)
            return carry

        lax.fori_loop(0, NCHUNK, chunk_body, 0)
        plsc.subcore_barrier()

        pltpu.sync_copy(accum.at[pl.ds(s * RPS, RPS)],
                        agg_out.at[pl.ds(c * N_PAD + s * RPS, RPS)])
        if with_deg:
            pltpu.sync_copy(dacc.at[pl.ds(s * RPS, RPS)],
                            deg_out.at[pl.ds(c * N_PAD + s * RPS, RPS)])


def _edge_agg(h, src_r, dst_r, zeros_pad, zeros1, with_deg):
    d = h.shape[1]
    out_type = [jax.ShapeDtypeStruct((NC * N_PAD, d), jnp.float32)]
    scratch = [
        pltpu.VMEM((NCHUNK, CHUNK), jnp.int32),
        pltpu.VMEM((NCHUNK, CHUNK), jnp.int32),
        pltpu.VMEM((CHUNK, d), jnp.float32),
    ]
    if with_deg:
        out_type.append(jax.ShapeDtypeStruct((NC * N_PAD,), jnp.float32))
        scratch.append(pltpu.VMEM((CHUNK,), jnp.float32))
    scratch.append(pltpu.VMEM_SHARED((N_PAD, d), jnp.float32))
    if with_deg:
        scratch.append(pltpu.VMEM_SHARED((N_PAD,), jnp.float32))
    scratch.append(pltpu.SemaphoreType.DMA)
    mesh = plsc.VectorSubcoreMesh(core_axis_name="c", subcore_axis_name="s")
    fn = pl.kernel(
        functools.partial(_edge_agg_body, with_deg, d),
        compiler_params=pltpu.CompilerParams(use_tc_tiling_on_sc=False),
        out_type=tuple(out_type),
        mesh=mesh,
        scratch_types=tuple(scratch),
    )
    return fn(h, src_r, dst_r, zeros_pad, zeros1)


def _combine0_body(p0, p1, h, w, b, d0, d1, h_out, degs_out):
    degs = d0[...] + d1[...] + 1.0
    degs_out[...] = degs
    agg = p0[...] + p1[...] + h[...]
    lin = jnp.dot(agg, w[...]) + b[...]
    h_out[...] = jnp.tanh(lin / degs)


def _combine_body(p0, p1, h, w, b, degs, h_out):
    agg = p0[...] + p1[...] + h[...]
    lin = jnp.dot(agg, w[...]) + b[...]
    h_out[...] = jnp.tanh(lin / degs[...])


def _combine0(p0, p1, h, w, b, d0, d1):
    return pl.pallas_call(
        _combine0_body,
        out_shape=(
            jax.ShapeDtypeStruct((N, w.shape[1]), jnp.float32),
            jax.ShapeDtypeStruct((N, 1), jnp.float32),
        ),
    )(p0, p1, h, w, b, d0, d1)


def _combine(p0, p1, h, w, b, degs):
    return pl.pallas_call(
        _combine_body,
        out_shape=jax.ShapeDtypeStruct((N, w.shape[1]), jnp.float32),
    )(p0, p1, h, w, b, degs)


def kernel(node_feat, edge_index, W0, b0, W1, b1, W2, b2, W3, b3, Wc1, bc1, Wc2, bc2, Wd, bd):
    src = edge_index[0]
    dst = edge_index[1]
    pad = E_PAD - E
    src_r = jnp.concatenate([src, jnp.zeros((pad,), jnp.int32)]).reshape(NW, NCHUNK, CHUNK)
    dst_r = jnp.concatenate([dst, jnp.full((pad,), N, jnp.int32)]).reshape(NW, NCHUNK, CHUNK)
    zeros128 = jnp.zeros((N_PAD, D), jnp.float32)
    zeros1 = jnp.zeros((N_PAD,), jnp.float32)

    h = node_feat
    degs = None
    cats = []
    for i, (W, b) in enumerate(((W0, b0), (W1, b1), (W2, b2), (W3, b3))):
        zp = zeros128[:, : h.shape[1]]
        if i == 0:
            aggp, degp = _edge_agg(h, src_r, dst_r, zp, zeros1, True)
            p0, p1 = aggp[:N], aggp[N_PAD : N_PAD + N]
            d0, d1 = degp[:N, None], degp[N_PAD : N_PAD + N, None]
            h, degs = _combine0(p0, p1, h, W, b, d0, d1)
        else:
            (aggp,) = _edge_agg(h, src_r, dst_r, zp, zeros1, False)
            p0, p1 = aggp[:N], aggp[N_PAD : N_PAD + N]
            h = _combine(p0, p1, h, W, b, degs)
        cats.append(h)

    cm = jnp.concatenate(cats, axis=1)
    sort_channel = cm[:, -1].reshape(G, NPG)
    _, topk_idx = jax.lax.top_k(sort_channel, K)
    feats = cm.reshape(G, NPG, TL)
    pooled = jnp.take_along_axis(feats, topk_idx[:, :, None], axis=1)
    x = pooled.reshape(G, 1, K * TL)
    dn = ('NCH', 'OIH', 'NCH')
    y = jax.lax.conv_general_dilated(x, Wc1, (TL,), 'VALID', dimension_numbers=dn) + bc1[None, :, None]
    y = jax.nn.relu(y)
    y = jax.lax.reduce_window(y, -jnp.inf, jax.lax.max, (1, 1, 2), (1, 1, 2), 'VALID')
    y = jax.lax.conv_general_dilated(y, Wc2, (1,), 'VALID', dimension_numbers=dn) + bc2[None, :, None]
    y = jax.nn.relu(y)
    flat = y.reshape(G, -1)
    out = jax.nn.relu(flat @ Wd + bd)
    return jax.nn.relu(out)
